# baseline (speedup 1.0000x reference)
import functools

import jax
import jax.numpy as jnp
from jax import lax
from jax.experimental import pallas as pl
from jax.experimental.pallas import tpu as pltpu
from jax.experimental.pallas import tpu_sc as plsc

N = 10000
NP = 10240
E = 320000
NC, NS = 2, 16
NW = NC * NS
KD = 128
CHD = 80
K = 64
CH = 160
EPW = CH * K
EP = NW * EPW
RPT = NP // NS
ZR = 16
BM = 2048
GRID = NP // BM

_mesh = plsc.VectorSubcoreMesh(
    core_axis_name="c", subcore_axis_name="s", num_cores=NC, num_subcores=NS)


@functools.partial(
    pl.kernel,
    out_type=jax.ShapeDtypeStruct((NC * NP,), jnp.float32),
    mesh=_mesh,
    scratch_types=[
        pltpu.VMEM((CHD, KD), jnp.int32),
        pltpu.VMEM((KD,), jnp.float32),
        pltpu.VMEM((NP // NS,), jnp.float32),
        pltpu.VMEM_SHARED((NP,), jnp.float32),
    ],
)
def _sc_degree(dst_hbm, out_hbm, didx, ones, zbuf, dacc):
    cid = lax.axis_index("c")
    sid = lax.axis_index("s")
    wid = sid * NC + cid
    epr = NP // NS

    for i in range(KD // 16):
        ones[pl.ds(i * 16, 16)] = jnp.ones((16,), jnp.float32)
    for i in range(epr // 16):
        zbuf[pl.ds(i * 16, 16)] = jnp.zeros((16,), jnp.float32)

    pltpu.sync_copy(zbuf, dacc.at[pl.ds(sid * epr, epr)])
    pltpu.sync_copy(dst_hbm.at[wid], didx)
    plsc.subcore_barrier()

    def body(c, _):
        pltpu.sync_copy(ones, dacc.at[didx.at[c]], add=True)
        return _
    lax.fori_loop(0, CHD, body, None)
    plsc.subcore_barrier()
    pltpu.sync_copy(dacc.at[pl.ds(sid * epr, epr)],
                    out_hbm.at[pl.ds(cid * NP + sid * epr, epr)])


HCH = CH // 2


@functools.partial(
    pl.kernel,
    out_type=jax.ShapeDtypeStruct((NC, NP, 128), jnp.float32),
    mesh=_mesh,
    scratch_types=[
        pltpu.VMEM((HCH, K), jnp.int32),
        pltpu.VMEM((HCH, K), jnp.int32),
        pltpu.VMEM((K, 128), jnp.float32),
        pltpu.VMEM((K, 128), jnp.float32),
        pltpu.VMEM_SHARED((NP, 128), jnp.float32),
        pltpu.SemaphoreType.DMA,
        pltpu.SemaphoreType.DMA,
        pltpu.SemaphoreType.DMA,
        pltpu.SemaphoreType.DMA,
    ],
)
def _sc_aggregate(y_hbm, src_hbm, dst_hbm, out_hbm,
                  sidx, didx, rows0, rows1, acc, g0, g1, s0, s1):
    cid = lax.axis_index("c")
    sid = lax.axis_index("s")
    wid = sid * NC + cid

    def zfill(i, _):
        r = i // 8
        col = (i % 8) * 16
        rows0[r, pl.ds(col, 16)] = jnp.zeros((16,), jnp.float32)
        return _
    lax.fori_loop(0, K * 8, zfill, None)
    zbase = sid * (NP // NS)
    for rep in range(NP // NS // K):
        pltpu.sync_copy(rows0, acc.at[pl.ds(zbase + rep * K, K)])
    plsc.subcore_barrier()

    gdummy = y_hbm.at[pl.ds(0, K)]
    for p in range(2):
        pltpu.sync_copy(src_hbm.at[wid, p], sidx)
        pltpu.sync_copy(dst_hbm.at[wid, p], didx)
        pltpu.async_copy(y_hbm.at[sidx.at[0]], rows0, g0)
        pltpu.async_copy(y_hbm.at[sidx.at[1]], rows1, g1)

        def body(g, _):
            c = g * 2
            pltpu.make_async_copy(gdummy, rows0, g0).wait()
            pltpu.async_copy(rows0, acc.at[didx.at[c]], s0, add=True)
            pltpu.make_async_copy(gdummy, rows1, g1).wait()
            pltpu.async_copy(rows1, acc.at[didx.at[c + 1]], s1, add=True)
            pltpu.make_async_copy(rows0, acc.at[didx.at[c]], s0).wait()

            @pl.when(c + 2 < HCH)
            def _():
                pltpu.async_copy(y_hbm.at[sidx.at[c + 2]], rows0, g0)

            pltpu.make_async_copy(rows1, acc.at[didx.at[c + 1]], s1).wait()

            @pl.when(c + 3 < HCH)
            def _():
                pltpu.async_copy(y_hbm.at[sidx.at[c + 3]], rows1, g1)
            return _
        lax.fori_loop(0, HCH // 2, body, None)
    plsc.subcore_barrier()
    pltpu.sync_copy(acc.at[pl.ds(sid * RPT, RPT)],
                    out_hbm.at[cid, pl.ds(sid * RPT, RPT)])


def _tc_first_body(degp_ref, x_ref, w_ref, dinv_ref, y_ref):
    deg = degp_ref[0] + degp_ref[1] + 1.0
    dinv = lax.rsqrt(deg)
    dinv_ref[...] = dinv
    y_ref[...] = dinv * jnp.dot(x_ref[...], w_ref[...],
                                preferred_element_type=jnp.float32)


def _tc_first(degp, x_pad, w):
    return pl.pallas_call(
        _tc_first_body,
        grid=(GRID,),
        in_specs=[
            pl.BlockSpec((NC, BM, 1), lambda i: (0, i, 0)),
            pl.BlockSpec((BM, 128), lambda i: (i, 0)),
            pl.BlockSpec((128, 128), lambda i: (0, 0)),
        ],
        out_specs=[
            pl.BlockSpec((BM, 1), lambda i: (i, 0)),
            pl.BlockSpec((BM, 128), lambda i: (i, 0)),
        ],
        out_shape=[
            jax.ShapeDtypeStruct((NP, 1), jnp.float32),
            jax.ShapeDtypeStruct((NP, 128), jnp.float32),
        ],
    )(degp, x_pad, w)


def _tc_mid_body(sp_ref, y_ref, dinv_ref, b_ref, w_ref, ynext_ref, csum_ref):
    i = pl.program_id(0)
    rows = lax.broadcasted_iota(jnp.int32, (BM, 1), 0) + i * BM
    s = sp_ref[0] + sp_ref[1] + y_ref[...]
    h = jnp.maximum(s * dinv_ref[...] + b_ref[...], 0.0)
    h = jnp.where(rows < N, h, 0.0)
    psum = jnp.broadcast_to(jnp.sum(h, axis=0, keepdims=True), (8, 128))

    @pl.when(i == 0)
    def _():
        csum_ref[...] = psum

    @pl.when(i > 0)
    def _():
        csum_ref[...] += psum

    ynext_ref[...] = dinv_ref[...] * jnp.dot(
        h, w_ref[...], preferred_element_type=jnp.float32)


def _tc_mid(sp, y, dinv, b, w):
    return pl.pallas_call(
        _tc_mid_body,
        grid=(GRID,),
        in_specs=[
            pl.BlockSpec((NC, BM, 128), lambda i: (0, i, 0)),
            pl.BlockSpec((BM, 128), lambda i: (i, 0)),
            pl.BlockSpec((BM, 1), lambda i: (i, 0)),
            pl.BlockSpec((1, 128), lambda i: (0, 0)),
            pl.BlockSpec((128, 128), lambda i: (0, 0)),
        ],
        out_specs=[
            pl.BlockSpec((BM, 128), lambda i: (i, 0)),
            pl.BlockSpec((8, 128), lambda i: (0, 0)),
        ],
        out_shape=[
            jax.ShapeDtypeStruct((NP, 128), jnp.float32),
            jax.ShapeDtypeStruct((8, 128), jnp.float32),
        ],
    )(sp, y, dinv, b, w)


def _tc_last_body(sp_ref, y_ref, dinv_ref, b_ref, c1_ref, c2_ref,
                  a1_ref, a2_ref, a3_ref, l1b_ref, l2w_ref, l2b_ref,
                  out_ref, acc_ref):
    i = pl.program_id(0)
    rows = lax.broadcasted_iota(jnp.int32, (BM, 1), 0) + i * BM
    s = sp_ref[0] + sp_ref[1] + y_ref[...]
    h = jnp.maximum(s * dinv_ref[...] + b_ref[...], 0.0)
    h = jnp.where(rows < N, h, 0.0)
    psum = jnp.broadcast_to(jnp.sum(h, axis=0, keepdims=True), (8, 128))

    @pl.when(i == 0)
    def _():
        acc_ref[...] = psum

    @pl.when(i > 0)
    def _():
        acc_ref[...] += psum

    @pl.when(i == GRID - 1)
    def _():
        inv = jnp.float32(1.0 / N)
        p1 = c1_ref[...] * inv
        p2 = c2_ref[...] * inv
        p3 = acc_ref[...] * inv
        t = (jnp.dot(p1, a1_ref[...], preferred_element_type=jnp.float32)
             + jnp.dot(p2, a2_ref[...], preferred_element_type=jnp.float32)
             + jnp.dot(p3, a3_ref[...], preferred_element_type=jnp.float32)
             + l1b_ref[...])
        t = jnp.maximum(t, 0.0)
        o = jnp.dot(t, l2w_ref[...], preferred_element_type=jnp.float32) \
            + l2b_ref[...]
        out_ref[...] = o[0:1, 0:16]


def _tc_last(sp, y, dinv, b, c1, c2, a1, a2, a3, l1b, l2w_pad, l2b_pad):
    return pl.pallas_call(
        _tc_last_body,
        grid=(GRID,),
        in_specs=[
            pl.BlockSpec((NC, BM, 128), lambda i: (0, i, 0)),
            pl.BlockSpec((BM, 128), lambda i: (i, 0)),
            pl.BlockSpec((BM, 1), lambda i: (i, 0)),
            pl.BlockSpec((1, 128), lambda i: (0, 0)),
            pl.BlockSpec((8, 128), lambda i: (0, 0)),
            pl.BlockSpec((8, 128), lambda i: (0, 0)),
            pl.BlockSpec((128, 128), lambda i: (0, 0)),
            pl.BlockSpec((128, 128), lambda i: (0, 0)),
            pl.BlockSpec((128, 128), lambda i: (0, 0)),
            pl.BlockSpec((1, 128), lambda i: (0, 0)),
            pl.BlockSpec((128, 128), lambda i: (0, 0)),
            pl.BlockSpec((1, 128), lambda i: (0, 0)),
        ],
        out_specs=pl.BlockSpec((1, 16), lambda i: (0, 0)),
        out_shape=jax.ShapeDtypeStruct((1, 16), jnp.float32),
        scratch_shapes=[pltpu.VMEM((8, 128), jnp.float32)],
    )(sp, y, dinv, b, c1, c2, a1, a2, a3, l1b, l2w_pad, l2b_pad)


def kernel(x, edge_index, batch, W1, b1, W2, b2, W3, b3,
           lin1_W, lin1_b, lin2_W, lin2_b):
    src = edge_index[0].astype(jnp.int32)
    dst = edge_index[1].astype(jnp.int32)
    pad = EP - E
    fill = jnp.arange(pad, dtype=jnp.int32)
    src_pad = jnp.concatenate([src, fill % 16])
    dst_pad = jnp.concatenate([dst, N + fill % (NP - N)])
    src3 = src_pad.reshape(NW, 2, HCH, K)
    dst3 = dst_pad.reshape(NW, 2, HCH, K)
    dst3d = dst_pad.reshape(NW, CHD, KD)
    x_pad = jnp.pad(x, ((0, NP - N), (0, 0)))

    degp = _sc_degree(dst3d).reshape(NC, NP, 1)
    dinv, y1 = _tc_first(degp, x_pad, W1)

    s1 = _sc_aggregate(y1, src3, dst3)
    y2, c1 = _tc_mid(s1, y1, dinv, b1.reshape(1, 128), W2)

    s2 = _sc_aggregate(y2, src3, dst3)
    y3, c2 = _tc_mid(s2, y2, dinv, b2.reshape(1, 128), W3)

    s3 = _sc_aggregate(y3, src3, dst3)
    a1, a2, a3 = (lin1_W[0:128], lin1_W[128:256], lin1_W[256:384])
    l2w_pad = jnp.pad(lin2_W, ((0, 0), (0, 112)))
    l2b_pad = jnp.pad(lin2_b, (0, 112)).reshape(1, 128)
    out = _tc_last(s3, y3, dinv, b3.reshape(1, 128), c1, c2,
                   a1, a2, a3, lin1_b.reshape(1, 128), l2w_pad, l2b_pad)
    return out


# speedup vs baseline: 1.2663x; 1.2663x over previous
import functools

import jax
import jax.numpy as jnp
from jax import lax
from jax.experimental import pallas as pl
from jax.experimental.pallas import tpu as pltpu
from jax.experimental.pallas import tpu_sc as plsc

N = 10000
NP = 10240
E = 320000
NC, NS = 2, 16
NW = NC * NS
KD = 128
CHD = 80
K = 128
CH = 80
NPASS = 4
EPW = CH * K
EP = NW * EPW
RPT = NP // NS
ZR = 16
BM = 2048
GRID = NP // BM

_mesh = plsc.VectorSubcoreMesh(
    core_axis_name="c", subcore_axis_name="s", num_cores=NC, num_subcores=NS)


@functools.partial(
    pl.kernel,
    out_type=jax.ShapeDtypeStruct((NC * NP,), jnp.float32),
    mesh=_mesh,
    scratch_types=[
        pltpu.VMEM((CHD, KD), jnp.int32),
        pltpu.VMEM((KD,), jnp.float32),
        pltpu.VMEM((NP // NS,), jnp.float32),
        pltpu.VMEM_SHARED((NP,), jnp.float32),
    ],
)
def _sc_degree(dst_hbm, out_hbm, didx, ones, zbuf, dacc):
    cid = lax.axis_index("c")
    sid = lax.axis_index("s")
    wid = sid * NC + cid
    epr = NP // NS

    for i in range(KD // 16):
        ones[pl.ds(i * 16, 16)] = jnp.ones((16,), jnp.float32)
    for i in range(epr // 16):
        zbuf[pl.ds(i * 16, 16)] = jnp.zeros((16,), jnp.float32)

    pltpu.sync_copy(zbuf, dacc.at[pl.ds(sid * epr, epr)])
    pltpu.sync_copy(dst_hbm.at[wid], didx)
    plsc.subcore_barrier()

    def body(c, _):
        pltpu.sync_copy(ones, dacc.at[didx.at[c]], add=True)
        return _
    lax.fori_loop(0, CHD, body, None)
    plsc.subcore_barrier()
    pltpu.sync_copy(dacc.at[pl.ds(sid * epr, epr)],
                    out_hbm.at[pl.ds(cid * NP + sid * epr, epr)])


PCH = CH // NPASS


@functools.partial(
    pl.kernel,
    out_type=jax.ShapeDtypeStruct((NC, NP, 128), jnp.float32),
    mesh=_mesh,
    scratch_types=[
        pltpu.VMEM((PCH, K), jnp.int32),
        pltpu.VMEM((PCH, K), jnp.int32),
        pltpu.VMEM((K, 128), jnp.float32),
        pltpu.VMEM((K, 128), jnp.float32),
        pltpu.VMEM_SHARED((NP, 128), jnp.float32),
        pltpu.SemaphoreType.DMA,
        pltpu.SemaphoreType.DMA,
    ],
)
def _sc_aggregate(y_hbm, src_hbm, dst_hbm, out_hbm,
                  sidx, didx, rows0, rows1, acc, g0, g1):
    cid = lax.axis_index("c")
    sid = lax.axis_index("s")
    wid = sid * NC + cid

    def zfill(i, _):
        r = i // 8
        col = (i % 8) * 16
        rows0[r, pl.ds(col, 16)] = jnp.zeros((16,), jnp.float32)
        return _
    lax.fori_loop(0, K * 8, zfill, None)
    zbase = sid * (NP // NS)
    for rep in range(NP // NS // K):
        pltpu.sync_copy(rows0, acc.at[pl.ds(zbase + rep * K, K)])
    plsc.subcore_barrier()

    gdummy = y_hbm.at[pl.ds(0, K)]
    for p in range(NPASS):
        pltpu.sync_copy(src_hbm.at[wid, p], sidx)
        pltpu.sync_copy(dst_hbm.at[wid, p], didx)
        pltpu.async_copy(y_hbm.at[sidx.at[0]], rows0, g0)

        def body(g, _):
            c = g * 2
            pltpu.async_copy(y_hbm.at[sidx.at[c + 1]], rows1, g1)
            pltpu.make_async_copy(gdummy, rows0, g0).wait()
            pltpu.sync_copy(rows0, acc.at[didx.at[c]], add=True)

            @pl.when(c + 2 < PCH)
            def _():
                pltpu.async_copy(y_hbm.at[sidx.at[c + 2]], rows0, g0)

            pltpu.make_async_copy(gdummy, rows1, g1).wait()
            pltpu.sync_copy(rows1, acc.at[didx.at[c + 1]], add=True)
            return _
        lax.fori_loop(0, PCH // 2, body, None)
    plsc.subcore_barrier()
    pltpu.sync_copy(acc.at[pl.ds(sid * RPT, RPT)],
                    out_hbm.at[cid, pl.ds(sid * RPT, RPT)])


def _tc_first_body(degp_ref, x_ref, w_ref, dinv_ref, y_ref):
    deg = degp_ref[0] + degp_ref[1] + 1.0
    dinv = lax.rsqrt(deg)
    dinv_ref[...] = dinv
    y_ref[...] = dinv * jnp.dot(x_ref[...], w_ref[...],
                                preferred_element_type=jnp.float32)


def _tc_first(degp, x_pad, w):
    return pl.pallas_call(
        _tc_first_body,
        grid=(GRID,),
        in_specs=[
            pl.BlockSpec((NC, BM, 1), lambda i: (0, i, 0)),
            pl.BlockSpec((BM, 128), lambda i: (i, 0)),
            pl.BlockSpec((128, 128), lambda i: (0, 0)),
        ],
        out_specs=[
            pl.BlockSpec((BM, 1), lambda i: (i, 0)),
            pl.BlockSpec((BM, 128), lambda i: (i, 0)),
        ],
        out_shape=[
            jax.ShapeDtypeStruct((NP, 1), jnp.float32),
            jax.ShapeDtypeStruct((NP, 128), jnp.float32),
        ],
    )(degp, x_pad, w)


def _tc_mid_body(sp_ref, y_ref, dinv_ref, b_ref, w_ref, ynext_ref, csum_ref):
    i = pl.program_id(0)
    rows = lax.broadcasted_iota(jnp.int32, (BM, 1), 0) + i * BM
    s = sp_ref[0] + sp_ref[1] + y_ref[...]
    h = jnp.maximum(s * dinv_ref[...] + b_ref[...], 0.0)
    h = jnp.where(rows < N, h, 0.0)
    psum = jnp.broadcast_to(jnp.sum(h, axis=0, keepdims=True), (8, 128))

    @pl.when(i == 0)
    def _():
        csum_ref[...] = psum

    @pl.when(i > 0)
    def _():
        csum_ref[...] += psum

    ynext_ref[...] = dinv_ref[...] * jnp.dot(
        h, w_ref[...], preferred_element_type=jnp.float32)


def _tc_mid(sp, y, dinv, b, w):
    return pl.pallas_call(
        _tc_mid_body,
        grid=(GRID,),
        in_specs=[
            pl.BlockSpec((NC, BM, 128), lambda i: (0, i, 0)),
            pl.BlockSpec((BM, 128), lambda i: (i, 0)),
            pl.BlockSpec((BM, 1), lambda i: (i, 0)),
            pl.BlockSpec((1, 128), lambda i: (0, 0)),
            pl.BlockSpec((128, 128), lambda i: (0, 0)),
        ],
        out_specs=[
            pl.BlockSpec((BM, 128), lambda i: (i, 0)),
            pl.BlockSpec((8, 128), lambda i: (0, 0)),
        ],
        out_shape=[
            jax.ShapeDtypeStruct((NP, 128), jnp.float32),
            jax.ShapeDtypeStruct((8, 128), jnp.float32),
        ],
    )(sp, y, dinv, b, w)


def _tc_last_body(sp_ref, y_ref, dinv_ref, b_ref, c1_ref, c2_ref,
                  a1_ref, a2_ref, a3_ref, l1b_ref, l2w_ref, l2b_ref,
                  out_ref, acc_ref):
    i = pl.program_id(0)
    rows = lax.broadcasted_iota(jnp.int32, (BM, 1), 0) + i * BM
    s = sp_ref[0] + sp_ref[1] + y_ref[...]
    h = jnp.maximum(s * dinv_ref[...] + b_ref[...], 0.0)
    h = jnp.where(rows < N, h, 0.0)
    psum = jnp.broadcast_to(jnp.sum(h, axis=0, keepdims=True), (8, 128))

    @pl.when(i == 0)
    def _():
        acc_ref[...] = psum

    @pl.when(i > 0)
    def _():
        acc_ref[...] += psum

    @pl.when(i == GRID - 1)
    def _():
        inv = jnp.float32(1.0 / N)
        p1 = c1_ref[...] * inv
        p2 = c2_ref[...] * inv
        p3 = acc_ref[...] * inv
        t = (jnp.dot(p1, a1_ref[...], preferred_element_type=jnp.float32)
             + jnp.dot(p2, a2_ref[...], preferred_element_type=jnp.float32)
             + jnp.dot(p3, a3_ref[...], preferred_element_type=jnp.float32)
             + l1b_ref[...])
        t = jnp.maximum(t, 0.0)
        o = jnp.dot(t, l2w_ref[...], preferred_element_type=jnp.float32) \
            + l2b_ref[...]
        out_ref[...] = o[0:1, 0:16]


def _tc_last(sp, y, dinv, b, c1, c2, a1, a2, a3, l1b, l2w_pad, l2b_pad):
    return pl.pallas_call(
        _tc_last_body,
        grid=(GRID,),
        in_specs=[
            pl.BlockSpec((NC, BM, 128), lambda i: (0, i, 0)),
            pl.BlockSpec((BM, 128), lambda i: (i, 0)),
            pl.BlockSpec((BM, 1), lambda i: (i, 0)),
            pl.BlockSpec((1, 128), lambda i: (0, 0)),
            pl.BlockSpec((8, 128), lambda i: (0, 0)),
            pl.BlockSpec((8, 128), lambda i: (0, 0)),
            pl.BlockSpec((128, 128), lambda i: (0, 0)),
            pl.BlockSpec((128, 128), lambda i: (0, 0)),
            pl.BlockSpec((128, 128), lambda i: (0, 0)),
            pl.BlockSpec((1, 128), lambda i: (0, 0)),
            pl.BlockSpec((128, 128), lambda i: (0, 0)),
            pl.BlockSpec((1, 128), lambda i: (0, 0)),
        ],
        out_specs=pl.BlockSpec((1, 16), lambda i: (0, 0)),
        out_shape=jax.ShapeDtypeStruct((1, 16), jnp.float32),
        scratch_shapes=[pltpu.VMEM((8, 128), jnp.float32)],
    )(sp, y, dinv, b, c1, c2, a1, a2, a3, l1b, l2w_pad, l2b_pad)


def kernel(x, edge_index, batch, W1, b1, W2, b2, W3, b3,
           lin1_W, lin1_b, lin2_W, lin2_b):
    src = edge_index[0].astype(jnp.int32)
    dst = edge_index[1].astype(jnp.int32)
    pad = EP - E
    fill = jnp.arange(pad, dtype=jnp.int32)
    src_pad = jnp.concatenate([src, fill % 16])
    dst_pad = jnp.concatenate([dst, N + fill % (NP - N)])
    src3 = src_pad.reshape(NW, NPASS, PCH, K)
    dst3 = dst_pad.reshape(NW, NPASS, PCH, K)
    dst3d = dst_pad.reshape(NW, CHD, KD)
    x_pad = jnp.pad(x, ((0, NP - N), (0, 0)))

    degp = _sc_degree(dst3d).reshape(NC, NP, 1)
    dinv, y1 = _tc_first(degp, x_pad, W1)

    s1 = _sc_aggregate(y1, src3, dst3)
    y2, c1 = _tc_mid(s1, y1, dinv, b1.reshape(1, 128), W2)

    s2 = _sc_aggregate(y2, src3, dst3)
    y3, c2 = _tc_mid(s2, y2, dinv, b2.reshape(1, 128), W3)

    s3 = _sc_aggregate(y3, src3, dst3)
    a1, a2, a3 = (lin1_W[0:128], lin1_W[128:256], lin1_W[256:384])
    l2w_pad = jnp.pad(lin2_W, ((0, 0), (0, 112)))
    l2b_pad = jnp.pad(lin2_b, (0, 112)).reshape(1, 128)
    out = _tc_last(s3, y3, dinv, b3.reshape(1, 128), c1, c2,
                   a1, a2, a3, lin1_b.reshape(1, 128), l2w_pad, l2b_pad)
    return out
